# Initial kernel scaffold; baseline (speedup 1.0000x reference)
import jax
import jax.numpy as jnp
from jax.experimental import pallas as pl


def kernel(x, table):
    raise NotImplementedError("write your pallas kernel here")


# speedup vs baseline: 1.0343x; 1.0343x over previous
import functools

import jax
import jax.numpy as jnp
from jax import lax
from jax.experimental import pallas as pl
from jax.experimental.pallas import tpu as pltpu
from jax.experimental.pallas import tpu_sc as plsc

NC = 2
NS = 16
NW = NC * NS
VOCAB = 1000
D = 1000
B_TOTAL = 1024 * 50
B_PER_W = B_TOTAL // NW
CHUNK = 40
N_CHUNKS = B_PER_W // CHUNK

_mesh = plsc.VectorSubcoreMesh(core_axis_name="c", subcore_axis_name="s")


@jax.jit
def _gather(table, idx3):
    @functools.partial(
        pl.kernel,
        mesh=_mesh,
        out_type=jax.ShapeDtypeStruct((B_TOTAL, D), jnp.float32),
        compiler_params=pltpu.CompilerParams(use_tc_tiling_on_sc=False),
        scratch_types=[
            pltpu.VMEM((N_CHUNKS, CHUNK), jnp.int32),
            pltpu.VMEM((CHUNK, D), jnp.float32),
            pltpu.VMEM((CHUNK, D), jnp.float32),
            pltpu.SemaphoreType.DMA,
            pltpu.SemaphoreType.DMA,
        ],
    )
    def k(table_hbm, idx_hbm, out_hbm, idx_v, rows0, rows1, sem0, sem1):
        wid = lax.axis_index("s") * NC + lax.axis_index("c")
        pltpu.sync_copy(idx_hbm.at[wid], idx_v)
        base = wid * B_PER_W

        pltpu.async_copy(table_hbm.at[idx_v.at[0]], rows0, sem0)

        def wait_gather(buf, sem):
            pltpu.make_async_copy(table_hbm.at[pl.ds(0, CHUNK)], buf, sem).wait()

        @pl.loop(0, N_CHUNKS, step=2)
        def _(c):
            pltpu.async_copy(table_hbm.at[idx_v.at[c + 1]], rows1, sem1)
            wait_gather(rows0, sem0)
            pltpu.sync_copy(rows0, out_hbm.at[pl.ds(base + c * CHUNK, CHUNK)])

            @pl.when(c + 2 < N_CHUNKS)
            def _():
                pltpu.async_copy(table_hbm.at[idx_v.at[c + 2]], rows0, sem0)

            wait_gather(rows1, sem1)
            pltpu.sync_copy(
                rows1, out_hbm.at[pl.ds(base + (c + 1) * CHUNK, CHUNK)]
            )

    return k(table, idx3)


def kernel(x, table):
    idx3 = x.reshape(NW, N_CHUNKS, CHUNK)
    out = _gather(table, idx3)
    return out.reshape(x.shape[0], x.shape[1], VOCAB)


# speedup vs baseline: 1.0345x; 1.0002x over previous
import functools

import jax
import jax.numpy as jnp
from jax import lax
from jax.experimental import pallas as pl
from jax.experimental.pallas import tpu as pltpu
from jax.experimental.pallas import tpu_sc as plsc

NC = 2
NS = 16
NW = NC * NS
VOCAB = 1000
D = 1000
BATCH = 1024
SEQ = 50
B_PER_W = BATCH // NW

_mesh = plsc.VectorSubcoreMesh(core_axis_name="c", subcore_axis_name="s")


@jax.jit
def _gather(table, idx3):
    @functools.partial(
        pl.kernel,
        mesh=_mesh,
        out_type=jax.ShapeDtypeStruct((BATCH, SEQ, D), jnp.float32),
        compiler_params=pltpu.CompilerParams(use_tc_tiling_on_sc=False),
        scratch_types=[
            pltpu.VMEM((B_PER_W, SEQ), jnp.int32),
            pltpu.VMEM((SEQ, D), jnp.float32),
            pltpu.VMEM((SEQ, D), jnp.float32),
            pltpu.SemaphoreType.DMA,
            pltpu.SemaphoreType.DMA,
        ],
    )
    def k(table_hbm, idx_hbm, out_hbm, idx_v, rows0, rows1, sem0, sem1):
        wid = lax.axis_index("s") * NC + lax.axis_index("c")
        pltpu.sync_copy(idx_hbm.at[wid], idx_v)
        base = wid * B_PER_W

        pltpu.async_copy(table_hbm.at[idx_v.at[0]], rows0, sem0)

        def wait_gather(buf, sem):
            pltpu.make_async_copy(table_hbm.at[pl.ds(0, SEQ)], buf, sem).wait()

        @pl.loop(0, B_PER_W, step=2)
        def _(b):
            pltpu.async_copy(table_hbm.at[idx_v.at[b + 1]], rows1, sem1)
            wait_gather(rows0, sem0)
            pltpu.sync_copy(rows0, out_hbm.at[base + b])

            @pl.when(b + 2 < B_PER_W)
            def _():
                pltpu.async_copy(table_hbm.at[idx_v.at[b + 2]], rows0, sem0)

            wait_gather(rows1, sem1)
            pltpu.sync_copy(rows1, out_hbm.at[base + b + 1])

    return k(table, idx3)


def kernel(x, table):
    idx3 = x.reshape(NW, B_PER_W, SEQ)
    return _gather(table, idx3)


# speedup vs baseline: 1.4187x; 1.3713x over previous
import functools

import jax
import jax.numpy as jnp
from jax import lax
from jax.experimental import pallas as pl
from jax.experimental.pallas import tpu as pltpu
from jax.experimental.pallas import tpu_sc as plsc

NC = 2
NS = 16
NW = NC * NS
VOCAB = 1000
D = 1000
D_PAD = 1024
B_TOTAL = 1024 * 50
B_PER_W = B_TOTAL // NW
CHUNK = 40
N_CHUNKS = B_PER_W // CHUNK

_mesh = plsc.VectorSubcoreMesh(core_axis_name="c", subcore_axis_name="s")


@jax.jit
def _gather(table_pad, idx3):
    @functools.partial(
        pl.kernel,
        mesh=_mesh,
        out_type=jax.ShapeDtypeStruct((B_TOTAL, D_PAD), jnp.float32),
        scratch_types=[
            pltpu.VMEM((N_CHUNKS, CHUNK), jnp.int32),
            pltpu.VMEM((CHUNK, D_PAD), jnp.float32),
            pltpu.VMEM((CHUNK, D_PAD), jnp.float32),
            pltpu.SemaphoreType.DMA,
            pltpu.SemaphoreType.DMA,
        ],
    )
    def k(table_hbm, idx_hbm, out_hbm, idx_v, rows0, rows1, sem0, sem1):
        wid = lax.axis_index("s") * NC + lax.axis_index("c")
        pltpu.sync_copy(idx_hbm.at[wid], idx_v)
        base = wid * B_PER_W

        pltpu.async_copy(table_hbm.at[idx_v.at[0]], rows0, sem0)

        def wait_gather(buf, sem):
            pltpu.make_async_copy(table_hbm.at[pl.ds(0, CHUNK)], buf, sem).wait()

        @pl.loop(0, N_CHUNKS, step=2)
        def _(c):
            pltpu.async_copy(table_hbm.at[idx_v.at[c + 1]], rows1, sem1)
            wait_gather(rows0, sem0)
            pltpu.sync_copy(rows0, out_hbm.at[pl.ds(base + c * CHUNK, CHUNK)])

            @pl.when(c + 2 < N_CHUNKS)
            def _():
                pltpu.async_copy(table_hbm.at[idx_v.at[c + 2]], rows0, sem0)

            wait_gather(rows1, sem1)
            pltpu.sync_copy(
                rows1, out_hbm.at[pl.ds(base + (c + 1) * CHUNK, CHUNK)]
            )

    return k(table_pad, idx3)


def kernel(x, table):
    table_pad = jnp.pad(table, ((0, 0), (0, D_PAD - D)))
    idx3 = x.reshape(NW, N_CHUNKS, CHUNK)
    out_pad = _gather(table_pad, idx3)
    return out_pad[:, :D].reshape(x.shape[0], x.shape[1], D)
